# baseline (device time: 55757 ns/iter reference)
import jax
import jax.numpy as jnp
from jax import lax
from jax.experimental import pallas as pl
from jax.experimental.pallas import tpu as pltpu

N_DEV = 16
N_Z = 4
N_Q = 4
NCHUNK = 8
N_ASEND = 3


def kernel(x, w_mat):
    m, k_per = x.shape
    k_per2, n = w_mat.shape
    assert k_per == k_per2
    m_blk = m // N_DEV
    wc = n // NCHUNK
    grp = N_Z * m_blk

    def body(x_ref, w_ref, out_ref,
             xb_ref, wb_ref, asend, acomm, ares, bcomm,
             a_ssem, asem, b_ssem, bsem):
        my = lax.axis_index("i")
        qq = my % N_Q
        zz = my // N_Q
        cw = zz * N_Q + (qq + 1) % N_Q
        ccw = zz * N_Q + (qq - 1) % N_Q

        barrier = pltpu.get_barrier_semaphore()
        nbrs = [cw, ccw] + [((zz + dz) % N_Z) * N_Q + qq for dz in (1, 2, 3)]
        for nbr in nbrs:
            pl.semaphore_signal(
                barrier, inc=1,
                device_id=(nbr,), device_id_type=pl.DeviceIdType.MESH,
            )
        xb_ref[:, :] = x_ref[:, :].astype(jnp.bfloat16)
        wb_ref[:, :] = w_ref[:, :].astype(jnp.bfloat16)
        pl.semaphore_wait(barrier, len(nbrs))

        def pblk(blk, k):
            rows = xb_ref[pl.ds(blk * m_blk, m_blk), :]
            return jnp.dot(rows, wb_ref[:, k * wc:(k + 1) * wc],
                           preferred_element_type=jnp.float32)

        a_descs = {}
        send_queue = []
        n_issue = [0]

        def a_hop(k, s):
            cwk = (k % 2 == 0)
            tgt = cw if cwk else ccw
            g = (qq - s - 1) % N_Q if cwk else (qq + s + 1) % N_Q
            c = n_issue[0]
            n_issue[0] += 1
            slot = c % N_ASEND
            if c >= N_ASEND:
                send_queue[c - N_ASEND].wait_send()
            if s > 0:
                a_descs[(k, s - 1)].wait_recv()
            for z2 in range(N_Z):
                piece = pblk(N_Q * z2 + g, k)
                if s > 0:
                    piece = piece + acomm[k, s - 1,
                                          z2 * m_blk:(z2 + 1) * m_blk, :]
                asend[slot, z2 * m_blk:(z2 + 1) * m_blk, :] = (
                    piece.astype(jnp.bfloat16))
            rdma = pltpu.make_async_remote_copy(
                src_ref=asend.at[slot],
                dst_ref=acomm.at[k, s],
                send_sem=a_ssem.at[slot],
                recv_sem=asem.at[k, s],
                device_id=(tgt,),
                device_id_type=pl.DeviceIdType.MESH,
            )
            rdma.start()
            a_descs[(k, s)] = rdma
            send_queue.append(rdma)

        def a_fin_b_start(k):
            a_descs[(k, N_Z - 2)].wait_recv()
            for z2 in range(N_Z):
                piece = pblk(N_Q * z2 + qq, k) \
                    + acomm[k, N_Z - 2, z2 * m_blk:(z2 + 1) * m_blk, :]
                ares[k, z2 * m_blk:(z2 + 1) * m_blk, :] = (
                    piece.astype(jnp.bfloat16))
            for t in range(N_Z - 1):
                zt = jnp.where(t >= zz, t + 1, t)
                rdma = pltpu.make_async_remote_copy(
                    src_ref=ares.at[k, pl.ds(zt * m_blk, m_blk)],
                    dst_ref=bcomm.at[k, zz],
                    send_sem=b_ssem.at[k, t],
                    recv_sem=bsem.at[k, zz],
                    device_id=(zt * N_Q + qq,),
                    device_id_type=pl.DeviceIdType.MESH,
                )
                rdma.start()

        for t in range(NCHUNK + N_Z - 1):
            for s in range(N_Z):
                k = t - s
                if 0 <= k < NCHUNK:
                    if s < N_Z - 1:
                        a_hop(k, s)
                    else:
                        a_fin_b_start(k)

        for k in range(NCHUNK):
            acc = ares[k, pl.ds(zz * m_blk, m_blk), :].astype(jnp.float32)
            for t in range(N_Z - 1):
                zs = jnp.where(t >= zz, t + 1, t)
                recv = pltpu.make_async_remote_copy(
                    src_ref=ares.at[k, pl.ds(zs * m_blk, m_blk)],
                    dst_ref=bcomm.at[k, zs],
                    send_sem=b_ssem.at[k, t],
                    recv_sem=bsem.at[k, zs],
                    device_id=(zs * N_Q + qq,),
                    device_id_type=pl.DeviceIdType.MESH,
                )
                recv.wait_recv()
                acc = acc + bcomm[k, zs].astype(jnp.float32)
            out_ref[:, k * wc:(k + 1) * wc] = jnp.maximum(acc, 0.0)

        for desc in send_queue[-N_ASEND:]:
            desc.wait_send()
        for k in range(NCHUNK):
            for t in range(N_Z - 1):
                zt = jnp.where(t >= zz, t + 1, t)
                dummy = pltpu.make_async_remote_copy(
                    src_ref=ares.at[k, pl.ds(zt * m_blk, m_blk)],
                    dst_ref=bcomm.at[k, zz],
                    send_sem=b_ssem.at[k, t],
                    recv_sem=bsem.at[k, zz],
                    device_id=(zt * N_Q + qq,),
                    device_id_type=pl.DeviceIdType.MESH,
                )
                dummy.wait_send()

    return pl.pallas_call(
        body,
        out_shape=jax.ShapeDtypeStruct((m_blk, n), jnp.float32),
        in_specs=[
            pl.BlockSpec(memory_space=pltpu.VMEM),
            pl.BlockSpec(memory_space=pltpu.VMEM),
        ],
        out_specs=pl.BlockSpec(memory_space=pltpu.VMEM),
        scratch_shapes=[
            pltpu.VMEM((m, k_per), jnp.bfloat16),
            pltpu.VMEM((k_per, n), jnp.bfloat16),
            pltpu.VMEM((N_ASEND, grp, wc), jnp.bfloat16),
            pltpu.VMEM((NCHUNK, N_Z - 1, grp, wc), jnp.bfloat16),
            pltpu.VMEM((NCHUNK, grp, wc), jnp.bfloat16),
            pltpu.VMEM((NCHUNK, N_Z, m_blk, wc), jnp.bfloat16),
            pltpu.SemaphoreType.DMA((N_ASEND,)),
            pltpu.SemaphoreType.DMA((NCHUNK, N_Z - 1)),
            pltpu.SemaphoreType.DMA((NCHUNK, N_Z - 1)),
            pltpu.SemaphoreType.DMA((NCHUNK, N_Z)),
        ],
        compiler_params=pltpu.CompilerParams(collective_id=0),
    )(x, w_mat)


# device time: 55130 ns/iter; 1.0114x vs baseline; 1.0114x over previous
import jax
import jax.numpy as jnp
from jax import lax
from jax.experimental import pallas as pl
from jax.experimental.pallas import tpu as pltpu

N_DEV = 16
N_Z = 4
N_Q = 4
NCHUNK = 8
N_ASEND = 6


def kernel(x, w_mat):
    m, k_per = x.shape
    k_per2, n = w_mat.shape
    assert k_per == k_per2
    m_blk = m // N_DEV
    wc = n // NCHUNK
    grp = N_Z * m_blk

    def body(x_ref, w_ref, out_ref,
             xb_ref, wb_ref, asend, acomm, ares, bcomm,
             a_ssem, asem, b_ssem, bsem):
        my = lax.axis_index("i")
        qq = my % N_Q
        zz = my // N_Q
        cw = zz * N_Q + (qq + 1) % N_Q
        ccw = zz * N_Q + (qq - 1) % N_Q

        barrier = pltpu.get_barrier_semaphore()
        nbrs = [cw, ccw] + [((zz + dz) % N_Z) * N_Q + qq for dz in (1, 2, 3)]
        for nbr in nbrs:
            pl.semaphore_signal(
                barrier, inc=1,
                device_id=(nbr,), device_id_type=pl.DeviceIdType.MESH,
            )
        xb_ref[:, :] = x_ref[:, :].astype(jnp.bfloat16)
        wb_ref[:, :] = w_ref[:, :].astype(jnp.bfloat16)
        pl.semaphore_wait(barrier, len(nbrs))

        def pblk(blk, k):
            rows = xb_ref[pl.ds(blk * m_blk, m_blk), :]
            return jnp.dot(rows, wb_ref[:, k * wc:(k + 1) * wc],
                           preferred_element_type=jnp.float32)

        a_descs = {}
        send_queue = []
        n_issue = [0]

        def a_hop(k, s):
            cwk = (k % 2 == 0)
            tgt = cw if cwk else ccw
            g = (qq - s - 1) % N_Q if cwk else (qq + s + 1) % N_Q
            c = n_issue[0]
            n_issue[0] += 1
            slot = c % N_ASEND
            if s > 0:
                a_descs[(k, s - 1)].wait_recv()
            if c >= N_ASEND:
                send_queue[c - N_ASEND].wait_send()
            for z2 in range(N_Z):
                piece = pblk(N_Q * z2 + g, k)
                if s > 0:
                    piece = piece + acomm[k, s - 1,
                                          z2 * m_blk:(z2 + 1) * m_blk, :]
                asend[slot, z2 * m_blk:(z2 + 1) * m_blk, :] = (
                    piece.astype(jnp.bfloat16))
            rdma = pltpu.make_async_remote_copy(
                src_ref=asend.at[slot],
                dst_ref=acomm.at[k, s],
                send_sem=a_ssem.at[slot],
                recv_sem=asem.at[k, s],
                device_id=(tgt,),
                device_id_type=pl.DeviceIdType.MESH,
            )
            rdma.start()
            a_descs[(k, s)] = rdma
            send_queue.append(rdma)

        def a_fin_b_start(k):
            a_descs[(k, N_Z - 2)].wait_recv()
            for z2 in range(N_Z):
                piece = pblk(N_Q * z2 + qq, k) \
                    + acomm[k, N_Z - 2, z2 * m_blk:(z2 + 1) * m_blk, :]
                ares[k, z2 * m_blk:(z2 + 1) * m_blk, :] = (
                    piece.astype(jnp.bfloat16))
            for t in range(N_Z - 1):
                zt = jnp.where(t >= zz, t + 1, t)
                rdma = pltpu.make_async_remote_copy(
                    src_ref=ares.at[k, pl.ds(zt * m_blk, m_blk)],
                    dst_ref=bcomm.at[k, zz],
                    send_sem=b_ssem.at[k, t],
                    recv_sem=bsem.at[k, zz],
                    device_id=(zt * N_Q + qq,),
                    device_id_type=pl.DeviceIdType.MESH,
                )
                rdma.start()

        for t in range(NCHUNK + N_Z - 1):
            for s in range(N_Z):
                k = t - s
                if 0 <= k < NCHUNK:
                    if s < N_Z - 1:
                        a_hop(k, s)
                    else:
                        a_fin_b_start(k)

        for k in range(NCHUNK):
            acc = ares[k, pl.ds(zz * m_blk, m_blk), :].astype(jnp.float32)
            for t in range(N_Z - 1):
                zs = jnp.where(t >= zz, t + 1, t)
                recv = pltpu.make_async_remote_copy(
                    src_ref=ares.at[k, pl.ds(zs * m_blk, m_blk)],
                    dst_ref=bcomm.at[k, zs],
                    send_sem=b_ssem.at[k, t],
                    recv_sem=bsem.at[k, zs],
                    device_id=(zs * N_Q + qq,),
                    device_id_type=pl.DeviceIdType.MESH,
                )
                recv.wait_recv()
                acc = acc + bcomm[k, zs].astype(jnp.float32)
            out_ref[:, k * wc:(k + 1) * wc] = jnp.maximum(acc, 0.0)

        for desc in send_queue[-N_ASEND:]:
            desc.wait_send()
        for k in range(NCHUNK):
            for t in range(N_Z - 1):
                zt = jnp.where(t >= zz, t + 1, t)
                dummy = pltpu.make_async_remote_copy(
                    src_ref=ares.at[k, pl.ds(zt * m_blk, m_blk)],
                    dst_ref=bcomm.at[k, zz],
                    send_sem=b_ssem.at[k, t],
                    recv_sem=bsem.at[k, zz],
                    device_id=(zt * N_Q + qq,),
                    device_id_type=pl.DeviceIdType.MESH,
                )
                dummy.wait_send()

    return pl.pallas_call(
        body,
        out_shape=jax.ShapeDtypeStruct((m_blk, n), jnp.float32),
        in_specs=[
            pl.BlockSpec(memory_space=pltpu.VMEM),
            pl.BlockSpec(memory_space=pltpu.VMEM),
        ],
        out_specs=pl.BlockSpec(memory_space=pltpu.VMEM),
        scratch_shapes=[
            pltpu.VMEM((m, k_per), jnp.bfloat16),
            pltpu.VMEM((k_per, n), jnp.bfloat16),
            pltpu.VMEM((N_ASEND, grp, wc), jnp.bfloat16),
            pltpu.VMEM((NCHUNK, N_Z - 1, grp, wc), jnp.bfloat16),
            pltpu.VMEM((NCHUNK, grp, wc), jnp.bfloat16),
            pltpu.VMEM((NCHUNK, N_Z, m_blk, wc), jnp.bfloat16),
            pltpu.SemaphoreType.DMA((N_ASEND,)),
            pltpu.SemaphoreType.DMA((NCHUNK, N_Z - 1)),
            pltpu.SemaphoreType.DMA((NCHUNK, N_Z - 1)),
            pltpu.SemaphoreType.DMA((NCHUNK, N_Z)),
        ],
        compiler_params=pltpu.CompilerParams(collective_id=0),
    )(x, w_mat)


# device time: 52355 ns/iter; 1.0650x vs baseline; 1.0530x over previous
import jax
import jax.numpy as jnp
from jax import lax
from jax.experimental import pallas as pl
from jax.experimental.pallas import tpu as pltpu

N_DEV = 16
N_PIECE = 4

PERM = [0, 4, 8, 12, 13, 9, 5, 1, 2, 6, 10, 14, 15, 11, 7, 3]
INV_PERM = [PERM.index(i) for i in range(N_DEV)]


def kernel(x, w_mat):
    m, k_per = x.shape
    k_per2, n = w_mat.shape
    assert k_per == k_per2
    m_blk = m // N_DEV
    nh = n // 2
    pw = nh // N_PIECE
    n_hop = N_DEV - 1

    def body(x_ref, w_ref, out_ref,
             xb_ref, wb_ref, send_r, send_l, comm_r, comm_l,
             ssem_r, ssem_l, rsem_r, rsem_l):
        my = lax.axis_index("i")

        def lut(idx, table):
            v = jnp.int32(table[0])
            for i in range(1, N_DEV):
                v = jnp.where(idx == i, jnp.int32(table[i]), v)
            return v

        r_pos = lut(my, INV_PERM)
        right = lut((r_pos + 1) % N_DEV, PERM)
        left = lut((r_pos - 1) % N_DEV, PERM)

        barrier = pltpu.get_barrier_semaphore()
        for nbr in (left, right):
            pl.semaphore_signal(
                barrier, inc=1,
                device_id=(nbr,), device_id_type=pl.DeviceIdType.MESH,
            )
        xb_ref[:, :] = x_ref[:, :].astype(jnp.bfloat16)
        wb_ref[:, :] = w_ref[:, :].astype(jnp.bfloat16)
        pl.semaphore_wait(barrier, 2)

        def partial(c, col0, width):
            rows = xb_ref[pl.ds(c * m_blk, m_blk), :]
            return jnp.dot(rows, wb_ref[:, col0:col0 + width],
                           preferred_element_type=jnp.float32)

        dirs = {
            "r": (send_r, comm_r, ssem_r, rsem_r, right),
            "l": (send_l, comm_l, ssem_l, rsem_l, left),
        }
        prev = {}

        for s in range(n_hop):
            c_r = lut((r_pos - s - 1) % N_DEV, PERM)
            c_l = lut((r_pos + s + 1) % N_DEV, PERM)
            for p in range(N_PIECE):
                for d in ("r", "l"):
                    sbuf, comm, ssem, rsem, tgt = dirs[d]
                    col0 = (0 if d == "r" else nh) + p * pw
                    c = c_r if d == "r" else c_l
                    piece = partial(c, col0, pw).astype(jnp.bfloat16)
                    if s > 0:
                        prev[(d, p)].wait()
                        piece = piece + comm[s - 1, p]
                    sbuf[p] = piece
                    rdma = pltpu.make_async_remote_copy(
                        src_ref=sbuf.at[p],
                        dst_ref=comm.at[s, p],
                        send_sem=ssem.at[p],
                        recv_sem=rsem.at[s, p],
                        device_id=(tgt,),
                        device_id_type=pl.DeviceIdType.MESH,
                    )
                    rdma.start()
                    prev[(d, p)] = rdma

        for p in range(N_PIECE):
            for d in ("r", "l"):
                _, comm, _, _, _ = dirs[d]
                col0 = (0 if d == "r" else nh) + p * pw
                own = partial(my, col0, pw)
                prev[(d, p)].wait()
                acc = own + comm[n_hop - 1, p].astype(jnp.float32)
                out_ref[:, col0:col0 + pw] = jnp.maximum(acc, 0.0)

    return pl.pallas_call(
        body,
        out_shape=jax.ShapeDtypeStruct((m_blk, n), jnp.float32),
        in_specs=[
            pl.BlockSpec(memory_space=pltpu.VMEM),
            pl.BlockSpec(memory_space=pltpu.VMEM),
        ],
        out_specs=pl.BlockSpec(memory_space=pltpu.VMEM),
        scratch_shapes=[
            pltpu.VMEM((m, k_per), jnp.bfloat16),
            pltpu.VMEM((k_per, n), jnp.bfloat16),
            pltpu.VMEM((N_PIECE, m_blk, pw), jnp.bfloat16),
            pltpu.VMEM((N_PIECE, m_blk, pw), jnp.bfloat16),
            pltpu.VMEM((n_hop, N_PIECE, m_blk, pw), jnp.bfloat16),
            pltpu.VMEM((n_hop, N_PIECE, m_blk, pw), jnp.bfloat16),
            pltpu.SemaphoreType.DMA((N_PIECE,)),
            pltpu.SemaphoreType.DMA((N_PIECE,)),
            pltpu.SemaphoreType.DMA((n_hop, N_PIECE)),
            pltpu.SemaphoreType.DMA((n_hop, N_PIECE)),
        ],
        compiler_params=pltpu.CompilerParams(collective_id=0),
    )(x, w_mat)
